# baseline (speedup 1.0000x reference)
import functools

import jax
import jax.numpy as jnp
from jax import lax
from jax.experimental import pallas as pl
from jax.experimental.pallas import tpu as pltpu
from jax.experimental.pallas import tpu_sc as plsc

N = 10000
E = 320000
G = 128
D_IN = 9
D_H = 128

NC = 2
NS = 16
NW = NC * NS
B = 64
NBLK = 160
CH = 32
NBLK0 = 96
NBLK1 = 224
EPAD = NW * NBLK * B
TOTB = EPAD // B
NPAD = 10240
R_TEC = NPAD // NS
BLKN = 2048

_mesh = plsc.VectorSubcoreMesh(core_axis_name="c", subcore_axis_name="s")


def _sc_deg_body(dst_hbm, ew_hbm, out_hbm, dstv, ewv, zerov, acc):
    c = lax.axis_index("c")
    s = lax.axis_index("s")
    w = s * NC + c
    pltpu.sync_copy(dst_hbm.at[pl.ds(w * NBLK, NBLK)], dstv)
    pltpu.sync_copy(ew_hbm.at[pl.ds(w * NBLK, NBLK)], ewv)
    for i in range(R_TEC // 16):
        zerov[pl.ds(i * 16, 16)] = jnp.zeros((16,), jnp.float32)
    pltpu.sync_copy(zerov, acc.at[pl.ds(s * R_TEC, R_TEC)])
    plsc.subcore_barrier()

    def step(b, carry):
        pltpu.sync_copy(ewv.at[b], acc.at[dstv.at[b]], add=True)
        return carry

    lax.fori_loop(0, NBLK, step, 0)
    plsc.subcore_barrier()
    pltpu.sync_copy(acc.at[pl.ds(s * R_TEC, R_TEC)],
                    out_hbm.at[c, pl.ds(s * R_TEC, R_TEC)])


_sc_deg = pl.kernel(
    _sc_deg_body,
    out_type=jax.ShapeDtypeStruct((NC, NPAD), jnp.float32),
    mesh=_mesh,
    scratch_types=[
        pltpu.VMEM((NBLK, B), jnp.int32),
        pltpu.VMEM((NBLK, B), jnp.float32),
        pltpu.VMEM((R_TEC,), jnp.float32),
        pltpu.VMEM_SHARED((NPAD,), jnp.float32),
    ],
)


def _sc_agg_body(z_hbm, src_hbm, dst_hbm, ew_hbm, out_hbm,
                 srcv, dstv, ewv, r0, r1, r2, r3,
                 g0, g1, g2, g3, s0, s1, s2, s3, acc, *, d):
    c = lax.axis_index("c")
    s = lax.axis_index("s")
    w = s * NC + c
    rows = (r0, r1, r2, r3)
    gsem = (g0, g1, g2, g3)
    ssem = (s0, s1, s2, s3)

    def zfill(i, carry):
        for j in range(d // 16):
            r0[i, pl.ds(j * 16, 16)] = jnp.zeros((16,), jnp.float32)
        return carry

    lax.fori_loop(0, B, zfill, 0)
    for k in range(R_TEC // B):
        pltpu.sync_copy(r0, acc.at[pl.ds(s * R_TEC + k * B, B)])
    plsc.subcore_barrier()

    def scale(p, b):
        def escale(eb, c2):
            ews = ewv[b, pl.ds(eb * 16, 16)]
            for k in range(16):
                sc = ews[k]
                e = eb * 16 + k
                for j in range(d // 16):
                    rows[p][e, pl.ds(j * 16, 16)] = (
                        rows[p][e, pl.ds(j * 16, 16)] * sc)
            return c2

        lax.fori_loop(0, B // 16, escale, 0)

    def issue_g(p, b):
        pltpu.async_copy(z_hbm.at[srcv.at[b]], rows[p], gsem[p])

    def wait_g(p, b):
        pltpu.make_async_copy(z_hbm.at[srcv.at[b]], rows[p], gsem[p]).wait()

    def issue_s(p, b):
        pltpu.async_copy(rows[p], acc.at[dstv.at[b]], ssem[p], add=True)

    def wait_s(p, b):
        pltpu.make_async_copy(rows[p], acc.at[dstv.at[b]], ssem[p]).wait()

    start = jnp.where(c == 0, s * NBLK0, NS * NBLK0 + s * NBLK1)
    nch = jnp.where(c == 0, NBLK0 // CH, NBLK1 // CH)

    def chunk_body(ch, carry):
        base = start + ch * CH
        pltpu.sync_copy(src_hbm.at[pl.ds(base, CH)], srcv)
        pltpu.sync_copy(dst_hbm.at[pl.ds(base, CH)], dstv)
        pltpu.sync_copy(ew_hbm.at[pl.ds(base, CH)], ewv)
        issue_g(0, 0)
        issue_g(1, 1)

        def piter(i, c2):
            for k in range(4):
                b = 4 * i + k
                wait_g(k, b)
                scale(k, b)
                issue_s(k, b)
                wait_s(k, b)
                p = (k + 2) % 4
                if k < 2:
                    issue_g(p, b + 2)
                else:
                    @pl.when(i <= CH // 4 - 2)
                    def _():
                        issue_g(p, b + 2)
            return c2

        lax.fori_loop(0, CH // 4, piter, 0)
        return carry

    lax.fori_loop(0, nch, chunk_body, 0)

    plsc.subcore_barrier()
    for k in range(R_TEC // 128):
        pltpu.sync_copy(acc.at[pl.ds(s * R_TEC + k * 128, 128)],
                        out_hbm.at[c, pl.ds(s * R_TEC + k * 128, 128)])


def _make_sc_agg(d):
    return pl.kernel(
        functools.partial(_sc_agg_body, d=d),
        out_type=jax.ShapeDtypeStruct((NC, NPAD, d), jnp.float32),
        mesh=_mesh,
        scratch_types=[
            pltpu.VMEM((CH, B), jnp.int32),
            pltpu.VMEM((CH, B), jnp.int32),
            pltpu.VMEM((CH, B), jnp.float32),
            pltpu.VMEM((B, d), jnp.float32),
            pltpu.VMEM((B, d), jnp.float32),
            pltpu.VMEM((B, d), jnp.float32),
            pltpu.VMEM((B, d), jnp.float32),
            pltpu.SemaphoreType.DMA,
            pltpu.SemaphoreType.DMA,
            pltpu.SemaphoreType.DMA,
            pltpu.SemaphoreType.DMA,
            pltpu.SemaphoreType.DMA,
            pltpu.SemaphoreType.DMA,
            pltpu.SemaphoreType.DMA,
            pltpu.SemaphoreType.DMA,
            pltpu.VMEM_SHARED((NPAD, d), jnp.float32),
        ],
    )


_sc_agg128 = _make_sc_agg(128)


def _tc_prep_body(degp_ref, xpad_ref, dinv_ref, z0_ref):
    deg = degp_ref[:, 0:1] + degp_ref[:, 1:2] + 1.0
    dinv = lax.rsqrt(deg)
    dinv_ref[...] = dinv
    z0_ref[...] = dinv * xpad_ref[...]


def _tc_prep_call(degp_t, xpad):
    return pl.pallas_call(
        _tc_prep_body,
        grid=(NPAD // BLKN,),
        in_specs=[
            pl.BlockSpec((BLKN, NC), lambda i: (i, 0)),
            pl.BlockSpec((BLKN, D_H), lambda i: (i, 0)),
        ],
        out_specs=[
            pl.BlockSpec((BLKN, 1), lambda i: (i, 0)),
            pl.BlockSpec((BLKN, D_H), lambda i: (i, 0)),
        ],
        out_shape=[
            jax.ShapeDtypeStruct((NPAD, 1), jnp.float32),
            jax.ShapeDtypeStruct((NPAD, D_H), jnp.float32),
        ],
    )(degp_t, xpad)


def _tc_layer_body(pp_ref, z_ref, dinv_ref, w_ref, b_ref, znext_ref):
    p = pp_ref[0] + pp_ref[1]
    u = dinv_ref[...] * (p + z_ref[...])
    h = jnp.dot(u, w_ref[...], preferred_element_type=jnp.float32) + b_ref[...]
    znext_ref[...] = dinv_ref[...] * jnp.maximum(h, 0.0)


def _tc_layer(pp, z, dinv, w, b2d):
    din = w.shape[0]
    return pl.pallas_call(
        _tc_layer_body,
        grid=(NPAD // BLKN,),
        in_specs=[
            pl.BlockSpec((NC, BLKN, din), lambda i: (0, i, 0)),
            pl.BlockSpec((BLKN, din), lambda i: (i, 0)),
            pl.BlockSpec((BLKN, 1), lambda i: (i, 0)),
            pl.BlockSpec((din, D_H), lambda i: (0, 0)),
            pl.BlockSpec((1, D_H), lambda i: (0, 0)),
        ],
        out_specs=pl.BlockSpec((BLKN, D_H), lambda i: (i, 0)),
        out_shape=jax.ShapeDtypeStruct((NPAD, D_H), jnp.float32),
    )(pp, z, dinv, w, b2d)


def _tc_final_body(pp_ref, z_ref, dinv_ref, w_ref, b_ref, batch_ref,
                   fw1_ref, fb1_ref, fw2_ref, fb2_ref, out_ref,
                   sums_acc, cnt_acc):
    i = pl.program_id(0)

    @pl.when(i == 0)
    def _():
        sums_acc[...] = jnp.zeros_like(sums_acc)
        cnt_acc[...] = jnp.zeros_like(cnt_acc)

    p = pp_ref[0] + pp_ref[1]
    u = dinv_ref[...] * (p + z_ref[...])
    h = jnp.dot(u, w_ref[...], preferred_element_type=jnp.float32) + b_ref[...]
    h = jnp.maximum(h, 0.0)
    ids = batch_ref[...]
    gid = lax.broadcasted_iota(jnp.int32, (G, BLKN), 0)
    onehot = jnp.where(gid == ids, 1.0, 0.0)
    sums_acc[...] += jnp.dot(onehot, h, preferred_element_type=jnp.float32)
    cnt_acc[...] += jnp.sum(onehot, axis=1, keepdims=True)

    @pl.when(i == pl.num_programs(0) - 1)
    def _():
        gx = sums_acc[...] / jnp.maximum(cnt_acc[...], 1.0)
        hid = jnp.dot(gx, fw1_ref[...], preferred_element_type=jnp.float32)
        hid = jnp.maximum(hid + fb1_ref[...], 0.0)
        out_ref[...] = (jnp.dot(hid, fw2_ref[...],
                                preferred_element_type=jnp.float32)
                        + fb2_ref[...])


def _tc_final(pp, z, dinv, w, b2d, batchp, fw1, fb1_2d, fw2p, fb2p):
    return pl.pallas_call(
        _tc_final_body,
        grid=(NPAD // BLKN,),
        in_specs=[
            pl.BlockSpec((NC, BLKN, D_H), lambda i: (0, i, 0)),
            pl.BlockSpec((BLKN, D_H), lambda i: (i, 0)),
            pl.BlockSpec((BLKN, 1), lambda i: (i, 0)),
            pl.BlockSpec((D_H, D_H), lambda i: (0, 0)),
            pl.BlockSpec((1, D_H), lambda i: (0, 0)),
            pl.BlockSpec((1, BLKN), lambda i: (0, i)),
            pl.BlockSpec((D_H, D_H), lambda i: (0, 0)),
            pl.BlockSpec((1, D_H), lambda i: (0, 0)),
            pl.BlockSpec((D_H, D_H), lambda i: (0, 0)),
            pl.BlockSpec((1, D_H), lambda i: (0, 0)),
        ],
        out_specs=pl.BlockSpec((G, D_H), lambda i: (0, 0)),
        out_shape=jax.ShapeDtypeStruct((G, D_H), jnp.float32),
        scratch_shapes=[
            pltpu.VMEM((G, D_H), jnp.float32),
            pltpu.VMEM((G, 1), jnp.float32),
        ],
    )(pp, z, dinv, w, b2d, batchp, fw1, fb1_2d, fw2p, fb2p)


def kernel(x, edge_index, edge_attr, batch, W1, b1, W2, b2, W3, b3,
           fW1, fb1, fW2, fb2):
    src = edge_index[0]
    dst = edge_index[1]
    pad = EPAD - E
    src3 = jnp.pad(src, (0, pad)).reshape(TOTB, B)
    dst3 = jnp.pad(dst, (0, pad)).reshape(TOTB, B)
    ew3 = jnp.pad(edge_attr, (0, pad)).reshape(TOTB, B)
    xpad = jnp.pad(x, ((0, NPAD - N), (0, D_H - D_IN)))
    batchp = jnp.pad(batch, (0, NPAD - N), constant_values=G).reshape(1, NPAD)
    w1p = jnp.pad(W1, ((0, D_H - D_IN), (0, 0)))
    fw2p = jnp.pad(fW2, ((0, 0), (0, D_H - 2)))
    fb2p = jnp.pad(fb2, (0, D_H - 2)).reshape(1, D_H)

    degp = _sc_deg(dst3, ew3)
    dinv, z0 = _tc_prep_call(degp.T, xpad)
    p1 = _sc_agg128(z0, src3, dst3, ew3)
    z1 = _tc_layer(p1, z0, dinv, w1p, b1.reshape(1, D_H))
    p2 = _sc_agg128(z1, src3, dst3, ew3)
    z2 = _tc_layer(p2, z1, dinv, W2, b2.reshape(1, D_H))
    p3 = _sc_agg128(z2, src3, dst3, ew3)
    pred = _tc_final(p3, z2, dinv, W3, b3.reshape(1, D_H), batchp,
                     fW1, fb1.reshape(1, D_H), fw2p, fb2p)
    return pred[:, :2]


# speedup vs baseline: 1.2083x; 1.2083x over previous
import functools

import jax
import jax.numpy as jnp
from jax import lax
from jax.experimental import pallas as pl
from jax.experimental.pallas import tpu as pltpu
from jax.experimental.pallas import tpu_sc as plsc

N = 10000
E = 320000
G = 128
D_IN = 9
D_H = 128

NC = 2
NS = 16
NW = NC * NS
B = 64
NBLK = 160
CH = 32
NBLK0 = 224
NBLK1 = 96
EPAD = NW * NBLK * B
TOTB = EPAD // B
NPAD = 10240
R_TEC = NPAD // NS
BLKN = 2048

_mesh = plsc.VectorSubcoreMesh(core_axis_name="c", subcore_axis_name="s")


def _sc_deg_body(dst_hbm, ew_hbm, out_hbm, dstv, ewv, zerov, acc):
    c = lax.axis_index("c")
    s = lax.axis_index("s")
    w = s * NC + c
    pltpu.sync_copy(dst_hbm.at[pl.ds(w * NBLK, NBLK)], dstv)
    pltpu.sync_copy(ew_hbm.at[pl.ds(w * NBLK, NBLK)], ewv)
    for i in range(R_TEC // 16):
        zerov[pl.ds(i * 16, 16)] = jnp.zeros((16,), jnp.float32)
    pltpu.sync_copy(zerov, acc.at[pl.ds(s * R_TEC, R_TEC)])
    plsc.subcore_barrier()

    def step(b, carry):
        pltpu.sync_copy(ewv.at[b], acc.at[dstv.at[b]], add=True)
        return carry

    lax.fori_loop(0, NBLK, step, 0)
    plsc.subcore_barrier()
    pltpu.sync_copy(acc.at[pl.ds(s * R_TEC, R_TEC)],
                    out_hbm.at[c, pl.ds(s * R_TEC, R_TEC)])


_sc_deg = pl.kernel(
    _sc_deg_body,
    out_type=jax.ShapeDtypeStruct((NC, NPAD), jnp.float32),
    mesh=_mesh,
    scratch_types=[
        pltpu.VMEM((NBLK, B), jnp.int32),
        pltpu.VMEM((NBLK, B), jnp.float32),
        pltpu.VMEM((R_TEC,), jnp.float32),
        pltpu.VMEM_SHARED((NPAD,), jnp.float32),
    ],
)


def _sc_agg_body(z_hbm, src_hbm, dst_hbm, ew_hbm, out_hbm,
                 srcv, dstv, ewv, r0, r1, r2, r3,
                 g0, g1, g2, g3, s0, s1, s2, s3, acc, *, d):
    c = lax.axis_index("c")
    s = lax.axis_index("s")
    w = s * NC + c
    rows = (r0, r1, r2, r3)
    gsem = (g0, g1, g2, g3)
    ssem = (s0, s1, s2, s3)

    def zfill(i, carry):
        for j in range(d // 16):
            r0[i, pl.ds(j * 16, 16)] = jnp.zeros((16,), jnp.float32)
        return carry

    lax.fori_loop(0, B, zfill, 0)
    for k in range(R_TEC // B):
        pltpu.sync_copy(r0, acc.at[pl.ds(s * R_TEC + k * B, B)])
    plsc.subcore_barrier()

    def scale(p, b):
        def escale(eb, c2):
            ews = ewv[b, pl.ds(eb * 16, 16)]
            for k in range(16):
                sc = ews[k]
                e = eb * 16 + k
                for j in range(d // 16):
                    rows[p][e, pl.ds(j * 16, 16)] = (
                        rows[p][e, pl.ds(j * 16, 16)] * sc)
            return c2

        lax.fori_loop(0, B // 16, escale, 0)

    def issue_g(p, b):
        pltpu.async_copy(z_hbm.at[srcv.at[b]], rows[p], gsem[p])

    def wait_g(p, b):
        pltpu.make_async_copy(z_hbm.at[srcv.at[b]], rows[p], gsem[p]).wait()

    def issue_s(p, b):
        pltpu.async_copy(rows[p], acc.at[dstv.at[b]], ssem[p], add=True)

    def wait_s(p, b):
        pltpu.make_async_copy(rows[p], acc.at[dstv.at[b]], ssem[p]).wait()

    start = jnp.where(c == 0, s * NBLK0, NS * NBLK0 + s * NBLK1)
    nch = jnp.where(c == 0, NBLK0 // CH, NBLK1 // CH)

    def chunk_body(ch, carry):
        base = start + ch * CH
        pltpu.sync_copy(src_hbm.at[pl.ds(base, CH)], srcv)
        pltpu.sync_copy(dst_hbm.at[pl.ds(base, CH)], dstv)
        pltpu.sync_copy(ew_hbm.at[pl.ds(base, CH)], ewv)
        issue_g(0, 0)
        issue_g(1, 1)

        def piter(i, c2):
            for k in range(4):
                b = 4 * i + k
                wait_g(k, b)
                scale(k, b)
                issue_s(k, b)
                wait_s(k, b)
                p = (k + 2) % 4
                if k < 2:
                    issue_g(p, b + 2)
                else:
                    @pl.when(i <= CH // 4 - 2)
                    def _():
                        issue_g(p, b + 2)
            return c2

        lax.fori_loop(0, CH // 4, piter, 0)
        return carry

    lax.fori_loop(0, nch, chunk_body, 0)

    plsc.subcore_barrier()
    for k in range(R_TEC // 128):
        pltpu.sync_copy(acc.at[pl.ds(s * R_TEC + k * 128, 128)],
                        out_hbm.at[c, pl.ds(s * R_TEC + k * 128, 128)])


def _make_sc_agg(d):
    return pl.kernel(
        functools.partial(_sc_agg_body, d=d),
        out_type=jax.ShapeDtypeStruct((NC, NPAD, d), jnp.float32),
        mesh=_mesh,
        scratch_types=[
            pltpu.VMEM((CH, B), jnp.int32),
            pltpu.VMEM((CH, B), jnp.int32),
            pltpu.VMEM((CH, B), jnp.float32),
            pltpu.VMEM((B, d), jnp.float32),
            pltpu.VMEM((B, d), jnp.float32),
            pltpu.VMEM((B, d), jnp.float32),
            pltpu.VMEM((B, d), jnp.float32),
            pltpu.SemaphoreType.DMA,
            pltpu.SemaphoreType.DMA,
            pltpu.SemaphoreType.DMA,
            pltpu.SemaphoreType.DMA,
            pltpu.SemaphoreType.DMA,
            pltpu.SemaphoreType.DMA,
            pltpu.SemaphoreType.DMA,
            pltpu.SemaphoreType.DMA,
            pltpu.VMEM_SHARED((NPAD, d), jnp.float32),
        ],
    )


_sc_agg128 = _make_sc_agg(128)


def _tc_prep_body(degp_ref, xpad_ref, dinv_ref, z0_ref):
    deg = degp_ref[:, 0:1] + degp_ref[:, 1:2] + 1.0
    dinv = lax.rsqrt(deg)
    dinv_ref[...] = dinv
    z0_ref[...] = dinv * xpad_ref[...]


def _tc_prep_call(degp_t, xpad):
    return pl.pallas_call(
        _tc_prep_body,
        grid=(NPAD // BLKN,),
        in_specs=[
            pl.BlockSpec((BLKN, NC), lambda i: (i, 0)),
            pl.BlockSpec((BLKN, D_H), lambda i: (i, 0)),
        ],
        out_specs=[
            pl.BlockSpec((BLKN, 1), lambda i: (i, 0)),
            pl.BlockSpec((BLKN, D_H), lambda i: (i, 0)),
        ],
        out_shape=[
            jax.ShapeDtypeStruct((NPAD, 1), jnp.float32),
            jax.ShapeDtypeStruct((NPAD, D_H), jnp.float32),
        ],
    )(degp_t, xpad)


def _tc_layer_body(pp_ref, z_ref, dinv_ref, w_ref, b_ref, znext_ref):
    p = pp_ref[0] + pp_ref[1]
    u = dinv_ref[...] * (p + z_ref[...])
    h = jnp.dot(u, w_ref[...], preferred_element_type=jnp.float32) + b_ref[...]
    znext_ref[...] = dinv_ref[...] * jnp.maximum(h, 0.0)


def _tc_layer(pp, z, dinv, w, b2d):
    din = w.shape[0]
    return pl.pallas_call(
        _tc_layer_body,
        grid=(NPAD // BLKN,),
        in_specs=[
            pl.BlockSpec((NC, BLKN, din), lambda i: (0, i, 0)),
            pl.BlockSpec((BLKN, din), lambda i: (i, 0)),
            pl.BlockSpec((BLKN, 1), lambda i: (i, 0)),
            pl.BlockSpec((din, D_H), lambda i: (0, 0)),
            pl.BlockSpec((1, D_H), lambda i: (0, 0)),
        ],
        out_specs=pl.BlockSpec((BLKN, D_H), lambda i: (i, 0)),
        out_shape=jax.ShapeDtypeStruct((NPAD, D_H), jnp.float32),
    )(pp, z, dinv, w, b2d)


def _tc_final_body(pp_ref, z_ref, dinv_ref, w_ref, b_ref, batch_ref,
                   fw1_ref, fb1_ref, fw2_ref, fb2_ref, out_ref,
                   sums_acc, cnt_acc):
    i = pl.program_id(0)

    @pl.when(i == 0)
    def _():
        sums_acc[...] = jnp.zeros_like(sums_acc)
        cnt_acc[...] = jnp.zeros_like(cnt_acc)

    p = pp_ref[0] + pp_ref[1]
    u = dinv_ref[...] * (p + z_ref[...])
    h = jnp.dot(u, w_ref[...], preferred_element_type=jnp.float32) + b_ref[...]
    h = jnp.maximum(h, 0.0)
    ids = batch_ref[...]
    gid = lax.broadcasted_iota(jnp.int32, (G, BLKN), 0)
    onehot = jnp.where(gid == ids, 1.0, 0.0)
    sums_acc[...] += jnp.dot(onehot, h, preferred_element_type=jnp.float32)
    cnt_acc[...] += jnp.sum(onehot, axis=1, keepdims=True)

    @pl.when(i == pl.num_programs(0) - 1)
    def _():
        gx = sums_acc[...] / jnp.maximum(cnt_acc[...], 1.0)
        hid = jnp.dot(gx, fw1_ref[...], preferred_element_type=jnp.float32)
        hid = jnp.maximum(hid + fb1_ref[...], 0.0)
        out_ref[...] = (jnp.dot(hid, fw2_ref[...],
                                preferred_element_type=jnp.float32)
                        + fb2_ref[...])


def _tc_final(pp, z, dinv, w, b2d, batchp, fw1, fb1_2d, fw2p, fb2p):
    return pl.pallas_call(
        _tc_final_body,
        grid=(NPAD // BLKN,),
        in_specs=[
            pl.BlockSpec((NC, BLKN, D_H), lambda i: (0, i, 0)),
            pl.BlockSpec((BLKN, D_H), lambda i: (i, 0)),
            pl.BlockSpec((BLKN, 1), lambda i: (i, 0)),
            pl.BlockSpec((D_H, D_H), lambda i: (0, 0)),
            pl.BlockSpec((1, D_H), lambda i: (0, 0)),
            pl.BlockSpec((1, BLKN), lambda i: (0, i)),
            pl.BlockSpec((D_H, D_H), lambda i: (0, 0)),
            pl.BlockSpec((1, D_H), lambda i: (0, 0)),
            pl.BlockSpec((D_H, D_H), lambda i: (0, 0)),
            pl.BlockSpec((1, D_H), lambda i: (0, 0)),
        ],
        out_specs=pl.BlockSpec((G, D_H), lambda i: (0, 0)),
        out_shape=jax.ShapeDtypeStruct((G, D_H), jnp.float32),
        scratch_shapes=[
            pltpu.VMEM((G, D_H), jnp.float32),
            pltpu.VMEM((G, 1), jnp.float32),
        ],
    )(pp, z, dinv, w, b2d, batchp, fw1, fb1_2d, fw2p, fb2p)


def kernel(x, edge_index, edge_attr, batch, W1, b1, W2, b2, W3, b3,
           fW1, fb1, fW2, fb2):
    src = edge_index[0]
    dst = edge_index[1]
    pad = EPAD - E
    src3 = jnp.pad(src, (0, pad)).reshape(TOTB, B)
    dst3 = jnp.pad(dst, (0, pad)).reshape(TOTB, B)
    ew3 = jnp.pad(edge_attr, (0, pad)).reshape(TOTB, B)
    xpad = jnp.pad(x, ((0, NPAD - N), (0, D_H - D_IN)))
    batchp = jnp.pad(batch, (0, NPAD - N), constant_values=G).reshape(1, NPAD)
    w1p = jnp.pad(W1, ((0, D_H - D_IN), (0, 0)))
    fw2p = jnp.pad(fW2, ((0, 0), (0, D_H - 2)))
    fb2p = jnp.pad(fb2, (0, D_H - 2)).reshape(1, D_H)

    degp = _sc_deg(dst3, ew3)
    dinv, z0 = _tc_prep_call(degp.T, xpad)
    p1 = _sc_agg128(z0, src3, dst3, ew3)
    z1 = _tc_layer(p1, z0, dinv, w1p, b1.reshape(1, D_H))
    p2 = _sc_agg128(z1, src3, dst3, ew3)
    z2 = _tc_layer(p2, z1, dinv, W2, b2.reshape(1, D_H))
    p3 = _sc_agg128(z2, src3, dst3, ew3)
    pred = _tc_final(p3, z2, dinv, W3, b3.reshape(1, D_H), batchp,
                     fW1, fb1.reshape(1, D_H), fw2p, fb2p)
    return pred[:, :2]
